# baseline (speedup 1.0000x reference)
import jax
import jax.numpy as jnp
from jax import lax
from jax.experimental import pallas as pl
from jax.experimental.pallas import tpu as pltpu
from jax.experimental.pallas import tpu_sc as plsc

E = 16384
NG = 16
MARGIN = 0.1
NC = 2
NS = 16
NW = NC * NS
L = 16
EPAD = E + 2 * L
SIGN_BIT = -2147483648


def _sc_body(o_hbm, y_hbm, eb_hbm, part_hbm, o_v, y_v, eb_v, gacc_v):
    wid = lax.axis_index("s") * NC + lax.axis_index("c")
    pltpu.sync_copy(o_hbm, o_v.at[pl.ds(0, E)])
    pltpu.sync_copy(y_hbm, y_v.at[pl.ds(0, E)])
    pltpu.sync_copy(eb_hbm, eb_v.at[pl.ds(0, E)])

    ends = []
    for g in range(NG):
        def bs_step(_, lohi, g=g):
            lo, hi = lohi
            mid = (lo + hi) // 2
            le = eb_v[pl.ds(mid, L)][0] <= g
            return jnp.where(le, mid + 1, lo), jnp.where(le, hi, mid)

        lo, _ = lax.fori_loop(0, 15, bs_step, (jnp.int32(0), jnp.int32(E)))
        ends.append(lo)

    lane = lax.iota(jnp.int32, L)
    zero16 = jnp.zeros((L,), jnp.float32)

    def block_body(b, start, end):
        ib = b * L
        iv = ib + lane
        yiv = y_v[pl.ds(ib, L)]
        oiv = o_v[pl.ds(ib, L)]
        moi = MARGIN - oiv
        poi = MARGIN + oiv

        def pair_j(j):
            yjb = jnp.broadcast_to(y_v[pl.ds(j, L)][0], (L,))
            ojb = jnp.broadcast_to(o_v[pl.ds(j, L)][0], (L,))
            v = jnp.where(yiv > yjb, moi + ojb, jnp.float32(MARGIN))
            v = jnp.where(yiv < yjb, poi - ojb, v)
            return jnp.maximum(v, 0.0)

        jlo1 = jnp.maximum(ib, start) + 1
        jhi1 = jnp.minimum(ib + L, end)

        def ph1(j, a):
            return a + jnp.where(iv < j, pair_j(j), 0.0)

        a1 = lax.fori_loop(jlo1, jhi1, ph1, zero16)

        ph2_lo = ib + L
        ph2_hi = jnp.where((ib + L > start) & (ib < end), end, ph2_lo)

        def ph2(j, a):
            return a + pair_j(j)

        a2 = plsc.parallel_loop(ph2_lo, ph2_hi, step=1, unroll=4,
                                carry=zero16)(ph2)
        lane_valid = (iv >= start) & (iv < end)
        return jnp.where(lane_valid, a1 + a2, 0.0)

    for g in range(NG):
        start = jnp.int32(0) if g == 0 else ends[g - 1]
        end = ends[g]
        q0 = (start // L) // 64
        q1 = ((end + L - 1) // L + 63) // 64

        def q_step(q, acc, start=start, end=end):
            acc = acc + block_body(q * 64 + wid, start, end)
            acc = acc + block_body(q * 64 + (63 - wid), start, end)
            return acc

        acc_g = lax.fori_loop(q0, q1, q_step, zero16)
        gacc_v[pl.ds(g * L, L)] = acc_g

    pltpu.sync_copy(gacc_v, part_hbm.at[wid])


def _sc_partials(outputs, y, edges_batch):
    mesh = plsc.VectorSubcoreMesh(
        core_axis_name="c", subcore_axis_name="s",
        num_cores=NC, num_subcores=NS,
    )
    f = pl.kernel(
        _sc_body,
        out_type=jax.ShapeDtypeStruct((NW, NG * L), jnp.float32),
        mesh=mesh,
        scratch_types=[
            pltpu.VMEM((EPAD,), jnp.float32),
            pltpu.VMEM((EPAD,), jnp.float32),
            pltpu.VMEM((EPAD,), jnp.int32),
            pltpu.VMEM((NG * L,), jnp.float32),
        ],
    )
    return f(outputs, y, edges_batch)


def _finish_body(part_ref, eb_ref, out_ref):
    part = part_ref[...]
    eb = eb_ref[...]
    total = jnp.float32(0.0)
    for g in range(NG):
        n = jnp.sum((eb == g).astype(jnp.float32))
        cnt = n * (n - 1.0) * 0.5
        s = jnp.sum(part[:, g * L:(g + 1) * L])
        total = total + s / jnp.maximum(cnt, 1.0)
    num_graphs = jnp.max(eb).astype(jnp.float32) + 1.0
    out_ref[...] = (total / num_graphs).reshape(1, 1)


@jax.jit
def kernel(outputs, y, edges_batch):
    part = _sc_partials(outputs, y, edges_batch)
    eb2d = edges_batch.reshape(128, 128)
    out = pl.pallas_call(
        _finish_body,
        out_shape=jax.ShapeDtypeStruct((1, 1), jnp.float32),
    )(part, eb2d)
    return out[0, 0]


# speedup vs baseline: 1.4250x; 1.4250x over previous
import jax
import jax.numpy as jnp
from jax import lax
from jax.experimental import pallas as pl
from jax.experimental.pallas import tpu as pltpu
from jax.experimental.pallas import tpu_sc as plsc

E = 16384
NG = 16
MARGIN = 0.1
NC = 2
NS = 16
NW = NC * NS
L = 16
EPAD = E + 2 * L
SIGN_BIT = -2147483648


def _sc_body(o_hbm, y_hbm, eb_hbm, part_hbm, o_v, y_v, eb_v, gacc_v):
    wid = lax.axis_index("s") * NC + lax.axis_index("c")
    pltpu.sync_copy(o_hbm, o_v.at[pl.ds(0, E)])
    pltpu.sync_copy(y_hbm, y_v.at[pl.ds(0, E)])
    pltpu.sync_copy(eb_hbm, eb_v.at[pl.ds(0, E)])

    ends = []
    for g in range(NG):
        def bs_step(_, lohi, g=g):
            lo, hi = lohi
            mid = (lo + hi) // 2
            le = eb_v[pl.ds(mid, L)][0] <= g
            return jnp.where(le, mid + 1, lo), jnp.where(le, hi, mid)

        lo, _ = lax.fori_loop(0, 15, bs_step, (jnp.int32(0), jnp.int32(E)))
        ends.append(lo)

    lane = lax.iota(jnp.int32, L)
    zero16 = jnp.zeros((L,), jnp.float32)

    def block_body(b, start, end):
        ib = b * L
        iv = ib + lane
        yiv = y_v[pl.ds(ib, L)]
        oiv = o_v[pl.ds(ib, L)]
        moi = MARGIN - oiv
        poi = MARGIN + oiv

        def pair_j(j):
            yjb = jnp.broadcast_to(y_v[pl.ds(j, L)][0], (L,))
            ojb = jnp.broadcast_to(o_v[pl.ds(j, L)][0], (L,))
            v = jnp.where(yiv > yjb, moi + ojb, jnp.float32(MARGIN))
            v = jnp.where(yiv < yjb, poi - ojb, v)
            return jnp.maximum(v, 0.0)

        jlo1 = jnp.maximum(ib, start) + 1
        jhi1 = jnp.minimum(ib + L, end)

        def ph1(j, a):
            return a + jnp.where(iv < j, pair_j(j), 0.0)

        a1 = lax.fori_loop(jlo1, jhi1, ph1, zero16)

        ph2_lo = ib + L
        ph2_hi = jnp.where((ib + L > start) & (ib < end), end, ph2_lo)
        hi_even = ph2_lo + ((ph2_hi - ph2_lo) & -2)

        def ph2(j, accs):
            a0, a1 = accs
            return a0 + pair_j(j), a1 + pair_j(j + 1)

        b0a, b1a = plsc.parallel_loop(
            ph2_lo, hi_even, step=2, unroll=4, carry=(zero16, zero16)
        )(ph2)
        hodd = jnp.where(hi_even < ph2_hi, jnp.float32(1.0), jnp.float32(0.0))
        a2 = b0a + b1a + pair_j(hi_even) * hodd
        lane_valid = (iv >= start) & (iv < end)
        return jnp.where(lane_valid, a1 + a2, 0.0)

    for g in range(NG):
        start = jnp.int32(0) if g == 0 else ends[g - 1]
        end = ends[g]
        q0 = (start // L) // 64
        q1 = ((end + L - 1) // L + 63) // 64

        def q_step(q, acc, start=start, end=end):
            acc = acc + block_body(q * 64 + wid, start, end)
            acc = acc + block_body(q * 64 + (63 - wid), start, end)
            return acc

        acc_g = lax.fori_loop(q0, q1, q_step, zero16)
        gacc_v[pl.ds(g * L, L)] = acc_g

    pltpu.sync_copy(gacc_v, part_hbm.at[wid])


def _sc_partials(outputs, y, edges_batch):
    mesh = plsc.VectorSubcoreMesh(
        core_axis_name="c", subcore_axis_name="s",
        num_cores=NC, num_subcores=NS,
    )
    f = pl.kernel(
        _sc_body,
        out_type=jax.ShapeDtypeStruct((NW, NG * L), jnp.float32),
        mesh=mesh,
        scratch_types=[
            pltpu.VMEM((EPAD,), jnp.float32),
            pltpu.VMEM((EPAD,), jnp.float32),
            pltpu.VMEM((EPAD,), jnp.int32),
            pltpu.VMEM((NG * L,), jnp.float32),
        ],
    )
    return f(outputs, y, edges_batch)


def _finish_body(part_ref, eb_ref, out_ref):
    part = part_ref[...]
    eb = eb_ref[...]
    total = jnp.float32(0.0)
    for g in range(NG):
        n = jnp.sum((eb == g).astype(jnp.float32))
        cnt = n * (n - 1.0) * 0.5
        s = jnp.sum(part[:, g * L:(g + 1) * L])
        total = total + s / jnp.maximum(cnt, 1.0)
    num_graphs = jnp.max(eb).astype(jnp.float32) + 1.0
    out_ref[...] = (total / num_graphs).reshape(1, 1)


@jax.jit
def kernel(outputs, y, edges_batch):
    part = _sc_partials(outputs, y, edges_batch)
    eb2d = edges_batch.reshape(128, 128)
    out = pl.pallas_call(
        _finish_body,
        out_shape=jax.ShapeDtypeStruct((1, 1), jnp.float32),
    )(part, eb2d)
    return out[0, 0]


# speedup vs baseline: 1.5611x; 1.0955x over previous
import jax
import jax.numpy as jnp
from jax import lax
from jax.experimental import pallas as pl
from jax.experimental.pallas import tpu as pltpu
from jax.experimental.pallas import tpu_sc as plsc

E = 16384
NG = 16
MARGIN = 0.1
NC = 2
NS = 16
NW = NC * NS
L = 16
EPAD = E + 2 * L
SIGN_BIT = -2147483648


def _sc_body(o_hbm, y_hbm, eb_hbm, part_hbm, o_v, y_v, eb_v, gacc_v):
    wid = lax.axis_index("s") * NC + lax.axis_index("c")
    pltpu.sync_copy(o_hbm, o_v.at[pl.ds(0, E)])
    pltpu.sync_copy(y_hbm, y_v.at[pl.ds(0, E)])
    pltpu.sync_copy(eb_hbm, eb_v.at[pl.ds(0, E)])

    ends = []
    for g in range(NG):
        def bs_step(_, lohi, g=g):
            lo, hi = lohi
            mid = (lo + hi) // 2
            le = eb_v[pl.ds(mid, L)][0] <= g
            return jnp.where(le, mid + 1, lo), jnp.where(le, hi, mid)

        lo, _ = lax.fori_loop(0, 15, bs_step, (jnp.int32(0), jnp.int32(E)))
        ends.append(lo)

    lane = lax.iota(jnp.int32, L)
    zero16 = jnp.zeros((L,), jnp.float32)

    def block_body(b, start, end):
        ib = b * L
        iv = ib + lane
        yiv = y_v[pl.ds(ib, L)]
        oiv = o_v[pl.ds(ib, L)]
        moi = MARGIN - oiv
        poi = MARGIN + oiv

        def pair_j(j):
            yjb = jnp.broadcast_to(y_v[pl.ds(j, L)][0], (L,))
            ojb = jnp.broadcast_to(o_v[pl.ds(j, L)][0], (L,))
            v = jnp.where(yiv > yjb, moi + ojb, jnp.float32(MARGIN))
            v = jnp.where(yiv < yjb, poi - ojb, v)
            return jnp.maximum(v, 0.0)

        jlo1 = jnp.maximum(ib, start) + 1
        jhi1 = jnp.minimum(ib + L, end)

        def ph1(j, a):
            return a + jnp.where(iv < j, pair_j(j), 0.0)

        a1 = lax.fori_loop(jlo1, jhi1, ph1, zero16)

        ph2_lo = ib + L
        ph2_hi = jnp.where((ib + L > start) & (ib < end), end, ph2_lo)
        hi_quad = ph2_lo + ((ph2_hi - ph2_lo) & -4)

        def ph2(j, accs):
            a0, a1, a2_, a3 = accs
            return (a0 + pair_j(j), a1 + pair_j(j + 1),
                    a2_ + pair_j(j + 2), a3 + pair_j(j + 3))

        b0a, b1a, b2a, b3a = plsc.parallel_loop(
            ph2_lo, hi_quad, step=4, unroll=2,
            carry=(zero16, zero16, zero16, zero16)
        )(ph2)
        a2 = b0a + b1a + b2a + b3a
        for t in range(3):
            ht = jnp.where(hi_quad + t < ph2_hi,
                           jnp.float32(1.0), jnp.float32(0.0))
            a2 = a2 + pair_j(hi_quad + t) * ht
        lane_valid = (iv >= start) & (iv < end)
        return jnp.where(lane_valid, a1 + a2, 0.0)

    for g in range(NG):
        start = jnp.int32(0) if g == 0 else ends[g - 1]
        end = ends[g]
        q0 = (start // L) // 64
        q1 = ((end + L - 1) // L + 63) // 64

        def q_step(q, acc, start=start, end=end):
            acc = acc + block_body(q * 64 + wid, start, end)
            acc = acc + block_body(q * 64 + (63 - wid), start, end)
            return acc

        acc_g = lax.fori_loop(q0, q1, q_step, zero16)
        gacc_v[pl.ds(g * L, L)] = acc_g

    pltpu.sync_copy(gacc_v, part_hbm.at[wid])


def _sc_partials(outputs, y, edges_batch):
    mesh = plsc.VectorSubcoreMesh(
        core_axis_name="c", subcore_axis_name="s",
        num_cores=NC, num_subcores=NS,
    )
    f = pl.kernel(
        _sc_body,
        out_type=jax.ShapeDtypeStruct((NW, NG * L), jnp.float32),
        mesh=mesh,
        scratch_types=[
            pltpu.VMEM((EPAD,), jnp.float32),
            pltpu.VMEM((EPAD,), jnp.float32),
            pltpu.VMEM((EPAD,), jnp.int32),
            pltpu.VMEM((NG * L,), jnp.float32),
        ],
    )
    return f(outputs, y, edges_batch)


def _finish_body(part_ref, eb_ref, out_ref):
    part = part_ref[...]
    eb = eb_ref[...]
    total = jnp.float32(0.0)
    for g in range(NG):
        n = jnp.sum((eb == g).astype(jnp.float32))
        cnt = n * (n - 1.0) * 0.5
        s = jnp.sum(part[:, g * L:(g + 1) * L])
        total = total + s / jnp.maximum(cnt, 1.0)
    num_graphs = jnp.max(eb).astype(jnp.float32) + 1.0
    out_ref[...] = (total / num_graphs).reshape(1, 1)


@jax.jit
def kernel(outputs, y, edges_batch):
    part = _sc_partials(outputs, y, edges_batch)
    eb2d = edges_batch.reshape(128, 128)
    out = pl.pallas_call(
        _finish_body,
        out_shape=jax.ShapeDtypeStruct((1, 1), jnp.float32),
    )(part, eb2d)
    return out[0, 0]
